# Initial kernel scaffold; baseline (speedup 1.0000x reference)
import jax
import jax.numpy as jnp
from jax.experimental import pallas as pl


def kernel(input_tensor, weight):
    raise NotImplementedError("write your pallas kernel here")


# speedup vs baseline: 1.5507x; 1.5507x over previous
import functools

import jax
import jax.numpy as jnp
from jax import lax
from jax.experimental import pallas as pl
from jax.experimental.pallas import tpu as pltpu
from jax.experimental.pallas import tpu_sc as plsc

_NC = 2
_NS = 16
_NW = _NC * _NS

_BATCH = 16384
_FIELDS = 26
_DIM = 32
_B = _BATCH * _FIELDS
_SUB = 128
_N_SUB = 8
_CHUNK = _SUB * _N_SUB
_B_PER_W = _B // _NW
_N_CHUNKS = _B_PER_W // _CHUNK


def _gather_body(tbl_hbm, idx_hbm, out_hbm, idx_v, rows_v, sem):
    wid = lax.axis_index("s") * _NC + lax.axis_index("c")
    idx_row0 = wid * (_B_PER_W // _SUB)

    def chunk_body(i, carry):
        row0 = idx_row0 + i * _N_SUB
        base = row0 * _SUB
        pltpu.sync_copy(idx_hbm.at[pl.ds(row0, _N_SUB)], idx_v)
        cps = [
            pltpu.async_copy(
                tbl_hbm.at[idx_v.at[j]],
                rows_v.at[pl.ds(j * _SUB, _SUB)],
                sem,
            )
            for j in range(_N_SUB)
        ]
        for cp in cps:
            cp.wait()
        pltpu.sync_copy(rows_v, out_hbm.at[pl.ds(base, _CHUNK)])
        return carry

    lax.fori_loop(0, _N_CHUNKS, chunk_body, 0)


_gather = pl.kernel(
    _gather_body,
    out_type=jax.ShapeDtypeStruct((_B, _DIM), jnp.float32),
    mesh=plsc.VectorSubcoreMesh(
        core_axis_name="c", subcore_axis_name="s",
        num_cores=_NC, num_subcores=_NS,
    ),
    scratch_types=[
        pltpu.VMEM((_N_SUB, _SUB), jnp.int32),
        pltpu.VMEM((_CHUNK, _DIM), jnp.float32),
        pltpu.SemaphoreType.DMA,
    ],
    compiler_params=pltpu.CompilerParams(use_tc_tiling_on_sc=False),
)


def kernel(input_tensor, weight):
    idx = input_tensor.reshape(-1).astype(jnp.int32).reshape(_B // _SUB, _SUB)
    out = _gather(weight, idx)
    return out.reshape(_BATCH, _FIELDS, _DIM)


# speedup vs baseline: 1.5691x; 1.0119x over previous
import jax
import jax.numpy as jnp
from jax import lax
from jax.experimental import pallas as pl
from jax.experimental.pallas import tpu as pltpu
from jax.experimental.pallas import tpu_sc as plsc

_NC = 2
_NS = 16
_NW = _NC * _NS

_BATCH = 16384
_FIELDS = 26
_DIM = 32
_B = _BATCH * _FIELDS
_ROWS_PER_W = _B // _NW
_CHUNK = 1024
_N_CHUNKS = _ROWS_PER_W // _CHUNK


def _gather_body(tbl_hbm, idx_hbm, out_hbm, idx_v, buf0, buf1, gsem0, gsem1,
                 ssem0, ssem1):
    wid = lax.axis_index("s") * _NC + lax.axis_index("c")
    base = wid * _ROWS_PER_W
    pltpu.sync_copy(idx_hbm.at[pl.ds(base, _ROWS_PER_W)], idx_v)

    bufs = (buf0, buf1)
    gsems = (gsem0, gsem1)
    ssems = (ssem0, ssem1)

    def fire_gather(i):
        p = i % 2
        return pltpu.async_copy(
            tbl_hbm.at[idx_v.at[pl.ds(i * _CHUNK, _CHUNK)]],
            bufs[p], gsems[p])

    def fire_store(i):
        p = i % 2
        return pltpu.async_copy(
            bufs[p], out_hbm.at[pl.ds(base + i * _CHUNK, _CHUNK)], ssems[p])

    gathers = [None] * _N_CHUNKS
    stores = [None] * _N_CHUNKS
    gathers[0] = fire_gather(0)
    for i in range(_N_CHUNKS):
        gathers[i].wait()
        if i >= 1:
            stores[i - 1].wait()
        if i + 1 < _N_CHUNKS:
            gathers[i + 1] = fire_gather(i + 1)
        stores[i] = fire_store(i)
    stores[_N_CHUNKS - 1].wait()


_gather = pl.kernel(
    _gather_body,
    out_type=jax.ShapeDtypeStruct((_B, _DIM), jnp.float32),
    mesh=plsc.VectorSubcoreMesh(
        core_axis_name="c", subcore_axis_name="s",
        num_cores=_NC, num_subcores=_NS,
    ),
    scratch_types=[
        pltpu.VMEM((_ROWS_PER_W,), jnp.int32),
        pltpu.VMEM((_CHUNK, _DIM), jnp.float32),
        pltpu.VMEM((_CHUNK, _DIM), jnp.float32),
        pltpu.SemaphoreType.DMA,
        pltpu.SemaphoreType.DMA,
        pltpu.SemaphoreType.DMA,
        pltpu.SemaphoreType.DMA,
    ],
    compiler_params=pltpu.CompilerParams(use_tc_tiling_on_sc=False),
)


def kernel(input_tensor, weight):
    idx = input_tensor.reshape(-1).astype(jnp.int32)
    out = _gather(weight, idx)
    return out.reshape(_BATCH, _FIELDS, _DIM)


# speedup vs baseline: 1.5755x; 1.0041x over previous
import jax
import jax.numpy as jnp
from jax import lax
from jax.experimental import pallas as pl
from jax.experimental.pallas import tpu as pltpu
from jax.experimental.pallas import tpu_sc as plsc

_NC = 2
_NS = 16
_NW = _NC * _NS

_BATCH = 16384
_FIELDS = 26
_DIM = 32
_B = _BATCH * _FIELDS
_ROWS_PER_W = _B // _NW
_CHUNK = 512
_NBUF = 4
_SPLIT = 4
_SUBC = _CHUNK // _SPLIT
_N_CHUNKS = _ROWS_PER_W // _CHUNK


def _gather_body(tbl_hbm, idx_hbm, out_hbm, idx_v, *rest):
    bufs = rest[:_NBUF]
    gsems = rest[_NBUF:2 * _NBUF]
    ssems = rest[2 * _NBUF:3 * _NBUF]

    wid = lax.axis_index("s") * _NC + lax.axis_index("c")
    base = wid * _ROWS_PER_W
    pltpu.sync_copy(idx_hbm.at[pl.ds(base, _ROWS_PER_W)], idx_v)

    def fire_gather(i):
        p = i % _NBUF
        return [
            pltpu.async_copy(
                tbl_hbm.at[idx_v.at[pl.ds(i * _CHUNK + k * _SUBC, _SUBC)]],
                bufs[p].at[pl.ds(k * _SUBC, _SUBC)], gsems[p])
            for k in range(_SPLIT)
        ]

    def fire_store(i):
        p = i % _NBUF
        return pltpu.async_copy(
            bufs[p], out_hbm.at[pl.ds(base + i * _CHUNK, _CHUNK)], ssems[p])

    gathers = [None] * _N_CHUNKS
    stores = [None] * _N_CHUNKS
    for i in range(min(_NBUF, _N_CHUNKS)):
        gathers[i] = fire_gather(i)
    for i in range(_N_CHUNKS):
        for cp in gathers[i]:
            cp.wait()
        stores[i] = fire_store(i)
        j = i + _NBUF
        if j < _N_CHUNKS:
            stores[i].wait()
            gathers[j] = fire_gather(j)
    for i in range(max(0, _N_CHUNKS - _NBUF), _N_CHUNKS):
        stores[i].wait()


_gather = pl.kernel(
    _gather_body,
    out_type=jax.ShapeDtypeStruct((_B, _DIM), jnp.float32),
    mesh=plsc.VectorSubcoreMesh(
        core_axis_name="c", subcore_axis_name="s",
        num_cores=_NC, num_subcores=_NS,
    ),
    scratch_types=(
        [pltpu.VMEM((_ROWS_PER_W,), jnp.int32)]
        + [pltpu.VMEM((_CHUNK, _DIM), jnp.float32) for _ in range(_NBUF)]
        + [pltpu.SemaphoreType.DMA for _ in range(2 * _NBUF)]
    ),
    compiler_params=pltpu.CompilerParams(use_tc_tiling_on_sc=False),
)


def kernel(input_tensor, weight):
    idx = input_tensor.reshape(-1).astype(jnp.int32)
    out = _gather(weight, idx)
    return out.reshape(_BATCH, _FIELDS, _DIM)
